# baseline (speedup 1.0000x reference)
import functools

import jax
import jax.numpy as jnp
from jax import lax
from jax.experimental import pallas as pl
from jax.experimental.pallas import tpu as pltpu
from jax.experimental.pallas import tpu_sc as plsc

N_NODES = 50000
N_EDGES = 800000
EDGES_PAD = 819200
_C = 16
_K = 640
_ACC_ROWS = 50176

_LAYER_DIMS = [(78, 96, 6, 64), (64, 64, 4, 128), (128, 128, 8, 256)]


def _eemb_body(ea_ref, wt_ref, b_ref, o_ref):
    o_ref[0] = (
        jnp.dot(ea_ref[...], wt_ref[0], preferred_element_type=jnp.float32)
        + b_ref[0]
    )


def _eemb(ea8, welt_pad, bel_pad, n_groups):
    ep = ea8.shape[0]
    be = 3200
    welt3 = jnp.transpose(welt_pad.reshape(8, n_groups, _C), (1, 0, 2))
    bel3 = bel_pad.reshape(n_groups, 1, _C)
    return pl.pallas_call(
        _eemb_body,
        grid=(n_groups, ep // be),
        in_specs=[
            pl.BlockSpec((be, 8), lambda g, e: (e, 0)),
            pl.BlockSpec((1, 8, _C), lambda g, e: (g, 0, 0)),
            pl.BlockSpec((1, 1, _C), lambda g, e: (g, 0, 0)),
        ],
        out_specs=pl.BlockSpec((1, be, _C), lambda g, e: (g, e, 0)),
        out_shape=jax.ShapeDtypeStruct((n_groups, ep, _C), jnp.float32),
    )(ea8, welt3, bel3)


_NR = 5
_NCH = 40


def _sc_aggr_body(n_groups, src_hbm, dst_hbm, *rest):
    xs = rest[:n_groups]
    es = rest[n_groups:2 * n_groups]
    outs = rest[2 * n_groups:3 * n_groups]
    (isrc0, isrc1, idst0, idst1, rows0, rows1, emb0, emb1, zbuf, accum,
     semg0, semg1, seme0, seme1, semsc0, semsc1) = rest[3 * n_groups:]
    sets = ((isrc0, idst0, rows0, emb0, semg0, seme0, semsc0),
            (isrc1, idst1, rows1, emb1, semg1, seme1, semsc1))
    c = lax.axis_index("c")
    s = lax.axis_index("s")

    zero16 = jnp.zeros((16,), jnp.float32)

    def zfill(i, carry):
        zbuf[i, pl.ds(0, 16)] = zero16
        return carry

    lax.fori_loop(0, 1024, zfill, 0, unroll=8)

    base128 = c * 3200 + s * 200

    for g in range(n_groups):
        xg, eg, out_g = xs[g], es[g], outs[g]
        r0 = s * 3136
        pltpu.sync_copy(zbuf, accum.at[pl.ds(r0, 1024)])
        pltpu.sync_copy(zbuf, accum.at[pl.ds(r0 + 1024, 1024)])
        pltpu.sync_copy(zbuf, accum.at[pl.ds(r0 + 2048, 1024)])
        pltpu.sync_copy(zbuf.at[pl.ds(0, 64)], accum.at[pl.ds(r0 + 3072, 64)])
        plsc.subcore_barrier()

        def fire(i, st):
            isrc, idst, rows, emb, semg, seme, _ = st
            rr = base128 + i * _NR
            e0 = rr * 128
            pltpu.async_copy(eg.at[pl.ds(e0, _K)], emb, seme)
            pltpu.sync_copy(src_hbm.at[pl.ds(e0, _K)], isrc)
            pltpu.sync_copy(dst_hbm.at[pl.ds(e0, _K)], idst)
            pltpu.async_copy(xg.at[isrc], rows, semg)

        def drain(sem, st):
            pltpu.make_async_copy(eg.at[pl.ds(0, _K)], st[3], sem).wait()

        def step(i, st, ot):
            isrc, idst, rows, emb, semg, seme, semsc = st

            drain(semg, st)
            drain(seme, st)

            def vbody(r, vc):
                a = pl.ds(0, 16)
                rows[r, a] = jnp.maximum(rows[r, a] + emb[r, a], 0.0)
                return vc

            lax.fori_loop(0, _K, vbody, 0, unroll=8)

            @pl.when(i >= 1)
            def _():
                drain(ot[6], ot)

            @pl.when(i + 1 < _NCH)
            def _():
                fire(i + 1, ot)

            pltpu.async_copy(rows, accum.at[idst], semsc, add=True)

        fire(0, sets[0])

        def pair(k, carry):
            step(2 * k, sets[0], sets[1])
            step(2 * k + 1, sets[1], sets[0])
            return carry

        lax.fori_loop(0, _NCH // 2, pair, 0)
        drain(sets[1][6], sets[1])
        plsc.subcore_barrier()

        w0 = s * 3128
        out_g = outs[g]

        @pl.when(s < 15)
        def _():
            pltpu.sync_copy(accum.at[pl.ds(w0, 3128)],
                            out_g.at[pl.ds(c * N_NODES + w0, 3128)])

        @pl.when(s == 15)
        def _():
            pltpu.sync_copy(accum.at[pl.ds(w0, 3080)],
                            out_g.at[pl.ds(c * N_NODES + w0, 3080)])

        plsc.subcore_barrier()


def _sc_aggr(src2d, dst2d, xs, es, n_groups):
    mesh = plsc.VectorSubcoreMesh(core_axis_name="c", subcore_axis_name="s")
    f = pl.kernel(
        functools.partial(_sc_aggr_body, n_groups),
        out_type=[jax.ShapeDtypeStruct((2 * N_NODES, _C), jnp.float32)
                  for _ in range(n_groups)],
        mesh=mesh,
        scratch_types=[
            pltpu.VMEM((_K,), jnp.int32),
            pltpu.VMEM((_K,), jnp.int32),
            pltpu.VMEM((_K,), jnp.int32),
            pltpu.VMEM((_K,), jnp.int32),
            pltpu.VMEM((_K, _C), jnp.float32),
            pltpu.VMEM((_K, _C), jnp.float32),
            pltpu.VMEM((_K, _C), jnp.float32),
            pltpu.VMEM((_K, _C), jnp.float32),
            pltpu.VMEM((1024, _C), jnp.float32),
            pltpu.VMEM_SHARED((_ACC_ROWS, _C), jnp.float32),
            pltpu.SemaphoreType.DMA,
            pltpu.SemaphoreType.DMA,
            pltpu.SemaphoreType.DMA,
            pltpu.SemaphoreType.DMA,
            pltpu.SemaphoreType.DMA,
            pltpu.SemaphoreType.DMA,
        ],
        compiler_params=pltpu.CompilerParams(use_tc_tiling_on_sc=False),
    )
    return f(src2d, dst2d, *xs, *es)


def _mlp_body(n_groups, x_ref, *refs, act, n_groups_next):
    parts = refs[:2 * n_groups]
    (w1_ref, b1_ref, w2_ref, b2_ref, g_ref, bt_ref, eps_ref) = \
        refs[2 * n_groups:2 * n_groups + 7]
    o_ref = refs[2 * n_groups + 7]
    o2_ref = refs[2 * n_groups + 8] if n_groups_next else None
    aggr = jnp.concatenate(
        [parts[2 * g][...] + parts[2 * g + 1][...] for g in range(n_groups)],
        axis=1)
    h0 = (1.0 + eps_ref[0, 0]) * x_ref[...] + aggr
    z = jnp.dot(h0, w1_ref[...], preferred_element_type=jnp.float32) + b1_ref[...]
    z = jnp.maximum(z, 0.0)
    h = jnp.dot(z, w2_ref[...], preferred_element_type=jnp.float32) + b2_ref[...]
    mu = jnp.mean(h, axis=-1, keepdims=True)
    var = jnp.mean((h - mu) ** 2, axis=-1, keepdims=True)
    h = (h - mu) * lax.rsqrt(var + 1e-5) * g_ref[...] + bt_ref[...]
    if act:
        h = jnp.maximum(h, 0.0)
    o_ref[...] = h
    if o2_ref is not None:
        br = h.shape[0]
        o2_ref[...] = jnp.transpose(
            h.reshape(br, n_groups_next, _C), (1, 0, 2))


def _mlp(xpad, parts, w1t_pad, b1, w2t, b2, gamma, beta, eps, act,
         n_groups_next):
    n, dp = xpad.shape
    n_groups = len(parts)
    dout = w2t.shape[0]
    br = 2000
    nblk = n // br
    out_shape = [jax.ShapeDtypeStruct((n, dout), jnp.float32)]
    out_specs = [pl.BlockSpec((br, dout), lambda i: (i, 0))]
    if n_groups_next:
        out_shape.append(
            jax.ShapeDtypeStruct((n_groups_next, n, _C), jnp.float32))
        out_specs.append(
            pl.BlockSpec((n_groups_next, br, _C), lambda i: (0, i, 0)))
    part_specs = []
    part_args = []
    for p in parts:
        part_specs.append(pl.BlockSpec((br, _C), lambda i: (i, 0)))
        part_specs.append(
            pl.BlockSpec((br, _C), lambda i, _nb=nblk: (_nb + i, 0)))
        part_args.extend([p, p])
    res = pl.pallas_call(
        functools.partial(_mlp_body, n_groups, act=act,
                          n_groups_next=n_groups_next),
        grid=(nblk,),
        in_specs=[
            pl.BlockSpec((br, dp), lambda i: (i, 0)),
            *part_specs,
            pl.BlockSpec((dp, dout), lambda i: (0, 0)),
            pl.BlockSpec((1, dout), lambda i: (0, 0)),
            pl.BlockSpec((dout, dout), lambda i: (0, 0)),
            pl.BlockSpec((1, dout), lambda i: (0, 0)),
            pl.BlockSpec((1, dout), lambda i: (0, 0)),
            pl.BlockSpec((1, dout), lambda i: (0, 0)),
            pl.BlockSpec((1, 1), lambda i: (0, 0), memory_space=pltpu.SMEM),
        ],
        out_specs=out_specs,
        out_shape=out_shape,
    )(xpad, *part_args, w1t_pad, b1[None, :], w2t, b2[None, :],
      gamma[None, :], beta[None, :], eps.reshape(1, 1))
    return res if n_groups_next else (res[0], None)


def kernel(x, edge_index, edge_attr, batch,
           l1_Wel, l1_bel, l1_W1, l1_b1, l1_W2, l1_b2, l1_gamma, l1_beta, l1_eps,
           l2_Wel, l2_bel, l2_W1, l2_b1, l2_W2, l2_b2, l2_gamma, l2_beta, l2_eps,
           l3_Wel, l3_bel, l3_W1, l3_b1, l3_W2, l3_b2, l3_gamma, l3_beta, l3_eps):
    layers = [
        (l1_Wel, l1_bel, l1_W1, l1_b1, l1_W2, l1_b2, l1_gamma, l1_beta, l1_eps, True),
        (l2_Wel, l2_bel, l2_W1, l2_b1, l2_W2, l2_b2, l2_gamma, l2_beta, l2_eps, True),
        (l3_Wel, l3_bel, l3_W1, l3_b1, l3_W2, l3_b2, l3_gamma, l3_beta, l3_eps, False),
    ]
    pad_e = EDGES_PAD - N_EDGES
    src2d = jnp.pad(edge_index[0], (0, pad_e))
    dst2d = jnp.pad(edge_index[1], (0, pad_e), constant_values=N_NODES)
    ea8 = jnp.pad(edge_attr, ((0, pad_e), (0, 2)))

    din1, dp1, p1, _ = _LAYER_DIMS[0]
    xpad = jnp.pad(x, ((0, 0), (0, dp1 - din1)))
    xsplit = jnp.transpose(xpad.reshape(N_NODES, p1, _C), (1, 0, 2))

    h = xpad
    for li, ((din, dp, ng, dout),
             (wel, bel, w1, b1, w2, b2, gamma, beta, eps, act)) in enumerate(
            zip(_LAYER_DIMS, layers)):
        welt_pad = jnp.pad(wel.T, ((0, 2), (0, dp - din)))
        bel_pad = jnp.pad(bel, (0, dp - din))[None, :]
        w1t_pad = jnp.pad(w1.T, ((0, dp - din), (0, 0)))
        eemb = _eemb(ea8, welt_pad, bel_pad, ng)
        parts = _sc_aggr(src2d, dst2d,
                         [xsplit[g] for g in range(ng)],
                         [eemb[g] for g in range(ng)], ng)
        ng_next = _LAYER_DIMS[li + 1][2] if li < 2 else 0
        h, xsplit = _mlp(h, parts, w1t_pad, b1, w2.T, b2, gamma, beta, eps,
                         act, ng_next)
    return (h, batch)


# speedup vs baseline: 2.0888x; 2.0888x over previous
import functools

import jax
import jax.numpy as jnp
from jax import lax
from jax.experimental import pallas as pl
from jax.experimental.pallas import tpu as pltpu
from jax.experimental.pallas import tpu_sc as plsc

N_NODES = 50000
N_EDGES = 800000
EDGES_PAD = 819200
_C = 16
_K = 640
_ACC_ROWS = 50176

_LAYER_DIMS = [(78, 96, 6, 64), (64, 64, 4, 128), (128, 128, 8, 256)]


def _eemb_body(ea_ref, wt_ref, b_ref, o_ref):
    o_ref[0] = (
        jnp.dot(ea_ref[...], wt_ref[0], preferred_element_type=jnp.float32)
        + b_ref[0]
    )


def _eemb(ea8w, welt_pad, bel_pad, n_groups):
    ep8 = ea8w.shape[0]
    be8 = 400
    eye8 = jnp.eye(8, dtype=jnp.float32)
    w8 = jnp.stack([
        jnp.kron(eye8, welt_pad[:, g * _C:(g + 1) * _C])
        for g in range(n_groups)])
    b128 = jnp.stack([
        jnp.tile(bel_pad[0, g * _C:(g + 1) * _C], 8)[None, :]
        for g in range(n_groups)])
    return pl.pallas_call(
        _eemb_body,
        grid=(n_groups, ep8 // be8),
        in_specs=[
            pl.BlockSpec((be8, 64), lambda g, e: (e, 0)),
            pl.BlockSpec((1, 64, 128), lambda g, e: (g, 0, 0)),
            pl.BlockSpec((1, 1, 128), lambda g, e: (g, 0, 0)),
        ],
        out_specs=pl.BlockSpec((1, be8, 128), lambda g, e: (g, e, 0)),
        out_shape=jax.ShapeDtypeStruct((n_groups, ep8, 128), jnp.float32),
    )(ea8w, w8, b128)


_NR = 5
_NCH = 40


def _sc_aggr_body(n_groups, src_hbm, dst_hbm, *rest):
    xs = rest[:n_groups]
    es = rest[n_groups:2 * n_groups]
    outs = rest[2 * n_groups:3 * n_groups]
    (isrc0, isrc1, idst0, idst1, rows0, rows1, emb0, emb1, zbuf, accum,
     semg0, semg1, seme0, seme1, semsc0, semsc1) = rest[3 * n_groups:]
    sets = ((isrc0, idst0, rows0, emb0, semg0, seme0, semsc0),
            (isrc1, idst1, rows1, emb1, semg1, seme1, semsc1))
    c = lax.axis_index("c")
    s = lax.axis_index("s")

    zero16 = jnp.zeros((16,), jnp.float32)

    def zfill(i, carry):
        zbuf[i, pl.ds(0, 16)] = zero16
        return carry

    lax.fori_loop(0, 1024, zfill, 0, unroll=8)

    base128 = c * 3200 + s * 200

    for g in range(n_groups):
        xg, eg, out_g = xs[g], es[g], outs[g]
        r0 = s * 3136
        pltpu.sync_copy(zbuf, accum.at[pl.ds(r0, 1024)])
        pltpu.sync_copy(zbuf, accum.at[pl.ds(r0 + 1024, 1024)])
        pltpu.sync_copy(zbuf, accum.at[pl.ds(r0 + 2048, 1024)])
        pltpu.sync_copy(zbuf.at[pl.ds(0, 64)], accum.at[pl.ds(r0 + 3072, 64)])
        plsc.subcore_barrier()

        def fire(i, st):
            isrc, idst, rows, emb, semg, seme, _ = st
            rr = base128 + i * _NR
            e0 = rr * 128
            pltpu.async_copy(eg.at[pl.ds(rr * 16, _K // 8)], emb, seme)
            pltpu.sync_copy(src_hbm.at[pl.ds(e0, _K)], isrc)
            pltpu.sync_copy(dst_hbm.at[pl.ds(e0, _K)], idst)
            pltpu.async_copy(xg.at[isrc], rows, semg)

        def drain(sem, st):
            pltpu.make_async_copy(eg.at[pl.ds(0, _K // 8)], st[3], sem).wait()

        def step(i, st, ot):
            isrc, idst, rows, emb, semg, seme, semsc = st

            drain(semg, st)
            drain(seme, st)

            def vbody(r8, vc):
                a = pl.ds(0, 16)
                for k in range(8):
                    rq = r8 * 8 + k
                    rows[rq, a] = jnp.maximum(
                        rows[rq, a] + emb[r8, pl.ds(k * 16, 16)], 0.0)
                return vc

            lax.fori_loop(0, _K // 8, vbody, 0, unroll=2)

            @pl.when(i >= 1)
            def _():
                drain(ot[6], ot)

            @pl.when(i + 1 < _NCH)
            def _():
                fire(i + 1, ot)

            pltpu.async_copy(rows, accum.at[idst], semsc, add=True)

        fire(0, sets[0])

        def pair(k, carry):
            step(2 * k, sets[0], sets[1])
            step(2 * k + 1, sets[1], sets[0])
            return carry

        lax.fori_loop(0, _NCH // 2, pair, 0)
        drain(sets[1][6], sets[1])
        plsc.subcore_barrier()

        w0 = s * 3128
        out_g = outs[g]

        @pl.when(s < 15)
        def _():
            pltpu.sync_copy(accum.at[pl.ds(w0, 3128)],
                            out_g.at[pl.ds(c * N_NODES + w0, 3128)])

        @pl.when(s == 15)
        def _():
            pltpu.sync_copy(accum.at[pl.ds(w0, 3080)],
                            out_g.at[pl.ds(c * N_NODES + w0, 3080)])

        plsc.subcore_barrier()


def _sc_aggr(src2d, dst2d, xs, es, n_groups):
    mesh = plsc.VectorSubcoreMesh(core_axis_name="c", subcore_axis_name="s")
    f = pl.kernel(
        functools.partial(_sc_aggr_body, n_groups),
        out_type=[jax.ShapeDtypeStruct((2 * N_NODES, _C), jnp.float32)
                  for _ in range(n_groups)],
        mesh=mesh,
        scratch_types=[
            pltpu.VMEM((_K,), jnp.int32),
            pltpu.VMEM((_K,), jnp.int32),
            pltpu.VMEM((_K,), jnp.int32),
            pltpu.VMEM((_K,), jnp.int32),
            pltpu.VMEM((_K, _C), jnp.float32),
            pltpu.VMEM((_K, _C), jnp.float32),
            pltpu.VMEM((_K // 8, 128), jnp.float32),
            pltpu.VMEM((_K // 8, 128), jnp.float32),
            pltpu.VMEM((1024, _C), jnp.float32),
            pltpu.VMEM_SHARED((_ACC_ROWS, _C), jnp.float32),
            pltpu.SemaphoreType.DMA,
            pltpu.SemaphoreType.DMA,
            pltpu.SemaphoreType.DMA,
            pltpu.SemaphoreType.DMA,
            pltpu.SemaphoreType.DMA,
            pltpu.SemaphoreType.DMA,
        ],
        compiler_params=pltpu.CompilerParams(use_tc_tiling_on_sc=False),
    )
    return f(src2d, dst2d, *xs, *es)


def _mlp_body(n_groups, x_ref, *refs, act, n_groups_next):
    parts = refs[:2 * n_groups]
    (w1_ref, b1_ref, w2_ref, b2_ref, g_ref, bt_ref, eps_ref) = \
        refs[2 * n_groups:2 * n_groups + 7]
    o_ref = refs[2 * n_groups + 7]
    o2_ref = refs[2 * n_groups + 8] if n_groups_next else None
    aggr = jnp.concatenate(
        [parts[2 * g][...] + parts[2 * g + 1][...] for g in range(n_groups)],
        axis=1)
    h0 = (1.0 + eps_ref[0, 0]) * x_ref[...] + aggr
    z = jnp.dot(h0, w1_ref[...], preferred_element_type=jnp.float32) + b1_ref[...]
    z = jnp.maximum(z, 0.0)
    h = jnp.dot(z, w2_ref[...], preferred_element_type=jnp.float32) + b2_ref[...]
    mu = jnp.mean(h, axis=-1, keepdims=True)
    var = jnp.mean((h - mu) ** 2, axis=-1, keepdims=True)
    h = (h - mu) * lax.rsqrt(var + 1e-5) * g_ref[...] + bt_ref[...]
    if act:
        h = jnp.maximum(h, 0.0)
    o_ref[...] = h
    if o2_ref is not None:
        br = h.shape[0]
        o2_ref[...] = jnp.transpose(
            h.reshape(br, n_groups_next, _C), (1, 0, 2))


def _mlp(xpad, parts, w1t_pad, b1, w2t, b2, gamma, beta, eps, act,
         n_groups_next):
    n, dp = xpad.shape
    n_groups = len(parts)
    dout = w2t.shape[0]
    br = 2000
    nblk = n // br
    out_shape = [jax.ShapeDtypeStruct((n, dout), jnp.float32)]
    out_specs = [pl.BlockSpec((br, dout), lambda i: (i, 0))]
    if n_groups_next:
        out_shape.append(
            jax.ShapeDtypeStruct((n_groups_next, n, _C), jnp.float32))
        out_specs.append(
            pl.BlockSpec((n_groups_next, br, _C), lambda i: (0, i, 0)))
    part_specs = []
    part_args = []
    for p in parts:
        part_specs.append(pl.BlockSpec((br, _C), lambda i: (i, 0)))
        part_specs.append(
            pl.BlockSpec((br, _C), lambda i, _nb=nblk: (_nb + i, 0)))
        part_args.extend([p, p])
    res = pl.pallas_call(
        functools.partial(_mlp_body, n_groups, act=act,
                          n_groups_next=n_groups_next),
        grid=(nblk,),
        in_specs=[
            pl.BlockSpec((br, dp), lambda i: (i, 0)),
            *part_specs,
            pl.BlockSpec((dp, dout), lambda i: (0, 0)),
            pl.BlockSpec((1, dout), lambda i: (0, 0)),
            pl.BlockSpec((dout, dout), lambda i: (0, 0)),
            pl.BlockSpec((1, dout), lambda i: (0, 0)),
            pl.BlockSpec((1, dout), lambda i: (0, 0)),
            pl.BlockSpec((1, dout), lambda i: (0, 0)),
            pl.BlockSpec((1, 1), lambda i: (0, 0), memory_space=pltpu.SMEM),
        ],
        out_specs=out_specs,
        out_shape=out_shape,
    )(xpad, *part_args, w1t_pad, b1[None, :], w2t, b2[None, :],
      gamma[None, :], beta[None, :], eps.reshape(1, 1))
    return res if n_groups_next else (res[0], None)


def kernel(x, edge_index, edge_attr, batch,
           l1_Wel, l1_bel, l1_W1, l1_b1, l1_W2, l1_b2, l1_gamma, l1_beta, l1_eps,
           l2_Wel, l2_bel, l2_W1, l2_b1, l2_W2, l2_b2, l2_gamma, l2_beta, l2_eps,
           l3_Wel, l3_bel, l3_W1, l3_b1, l3_W2, l3_b2, l3_gamma, l3_beta, l3_eps):
    layers = [
        (l1_Wel, l1_bel, l1_W1, l1_b1, l1_W2, l1_b2, l1_gamma, l1_beta, l1_eps, True),
        (l2_Wel, l2_bel, l2_W1, l2_b1, l2_W2, l2_b2, l2_gamma, l2_beta, l2_eps, True),
        (l3_Wel, l3_bel, l3_W1, l3_b1, l3_W2, l3_b2, l3_gamma, l3_beta, l3_eps, False),
    ]
    pad_e = EDGES_PAD - N_EDGES
    src2d = jnp.pad(edge_index[0], (0, pad_e))
    dst2d = jnp.pad(edge_index[1], (0, pad_e), constant_values=N_NODES)
    ea8w = jnp.pad(edge_attr, ((0, pad_e), (0, 2))).reshape(-1, 64)

    din1, dp1, p1, _ = _LAYER_DIMS[0]
    xpad = jnp.pad(x, ((0, 0), (0, dp1 - din1)))
    xsplit = jnp.transpose(xpad.reshape(N_NODES, p1, _C), (1, 0, 2))

    h = xpad
    for li, ((din, dp, ng, dout),
             (wel, bel, w1, b1, w2, b2, gamma, beta, eps, act)) in enumerate(
            zip(_LAYER_DIMS, layers)):
        welt_pad = jnp.pad(wel.T, ((0, 2), (0, dp - din)))
        bel_pad = jnp.pad(bel, (0, dp - din))[None, :]
        w1t_pad = jnp.pad(w1.T, ((0, dp - din), (0, 0)))
        eemb = _eemb(ea8w, welt_pad, bel_pad, ng)
        parts = _sc_aggr(src2d, dst2d,
                         [xsplit[g] for g in range(ng)],
                         [eemb[g] for g in range(ng)], ng)
        ng_next = _LAYER_DIMS[li + 1][2] if li < 2 else 0
        h, xsplit = _mlp(h, parts, w1t_pad, b1, w2.T, b2, gamma, beta, eps,
                         act, ng_next)
    return (h, batch)


# speedup vs baseline: 2.2138x; 1.0599x over previous
import functools

import jax
import jax.numpy as jnp
from jax import lax
from jax.experimental import pallas as pl
from jax.experimental.pallas import tpu as pltpu
from jax.experimental.pallas import tpu_sc as plsc

N_NODES = 50000
N_EDGES = 800000
EDGES_PAD = 819200
_C = 16
_K = 640
_ACC_ROWS = 50176

_LAYER_DIMS = [(78, 96, 6, 64), (64, 64, 4, 128), (128, 128, 8, 256)]


def _eemb_body(ea_ref, wt_ref, b_ref, o_ref):
    o_ref[0] = (
        jnp.dot(ea_ref[...], wt_ref[0], preferred_element_type=jnp.float32)
        + b_ref[0]
    )


def _eemb(ea8w, welt_pad, bel_pad, n_groups):
    ep8 = ea8w.shape[0]
    be8 = 400
    eye8 = jnp.eye(8, dtype=jnp.float32)
    w8 = jnp.stack([
        jnp.kron(eye8, welt_pad[:, g * _C:(g + 1) * _C])
        for g in range(n_groups)])
    b128 = jnp.stack([
        jnp.tile(bel_pad[0, g * _C:(g + 1) * _C], 8)[None, :]
        for g in range(n_groups)])
    return pl.pallas_call(
        _eemb_body,
        grid=(n_groups, ep8 // be8),
        in_specs=[
            pl.BlockSpec((be8, 64), lambda g, e: (e, 0)),
            pl.BlockSpec((1, 64, 128), lambda g, e: (g, 0, 0)),
            pl.BlockSpec((1, 1, 128), lambda g, e: (g, 0, 0)),
        ],
        out_specs=pl.BlockSpec((1, be8, 128), lambda g, e: (g, e, 0)),
        out_shape=jax.ShapeDtypeStruct((n_groups, ep8, 128), jnp.float32),
    )(ea8w, w8, b128)


_NR = 5
_NCH = 40


def _sc_aggr_body(n_groups, src_hbm, dst_hbm, *rest):
    xs = rest[:n_groups]
    es = rest[n_groups:2 * n_groups]
    outs = rest[2 * n_groups:3 * n_groups]
    (isrc0, isrc1, idst0, idst1, rows0, rows1, emb0, emb1, zbuf, accum,
     semg0, semg1, seme0, seme1, semsc0, semsc1) = rest[3 * n_groups:]
    sets = ((isrc0, idst0, rows0, emb0, semg0, seme0, semsc0),
            (isrc1, idst1, rows1, emb1, semg1, seme1, semsc1))
    c = lax.axis_index("c")
    s = lax.axis_index("s")

    zero16 = jnp.zeros((16,), jnp.float32)

    def zfill(i, carry):
        zbuf[i, pl.ds(0, 16)] = zero16
        return carry

    lax.fori_loop(0, 1024, zfill, 0, unroll=8)

    base128 = c * 3200 + s * 200

    for g in range(n_groups):
        xg, eg, out_g = xs[g], es[g], outs[g]
        r0 = s * 3136
        pltpu.sync_copy(zbuf, accum.at[pl.ds(r0, 1024)])
        pltpu.sync_copy(zbuf, accum.at[pl.ds(r0 + 1024, 1024)])
        pltpu.sync_copy(zbuf, accum.at[pl.ds(r0 + 2048, 1024)])
        pltpu.sync_copy(zbuf.at[pl.ds(0, 64)], accum.at[pl.ds(r0 + 3072, 64)])
        plsc.subcore_barrier()

        def fire(i, st):
            isrc, idst, rows, emb, semg, seme, _ = st
            rr = base128 + i * _NR
            e0 = rr * 128
            pltpu.async_copy(eg.at[pl.ds(rr * 16, _K // 8)], emb, seme)
            pltpu.sync_copy(src_hbm.at[pl.ds(e0, _K)], isrc)
            pltpu.sync_copy(dst_hbm.at[pl.ds(e0, _K)], idst)
            pltpu.async_copy(xg.at[isrc], rows, semg)

        def drain(sem, st):
            pltpu.make_async_copy(eg.at[pl.ds(0, _K // 8)], st[3], sem).wait()

        def step(i, st, ot):
            isrc, idst, rows, emb, semg, seme, semsc = st

            drain(semg, st)
            drain(seme, st)

            def vbody(r8, vc):
                a = pl.ds(0, 16)
                for k in range(8):
                    rq = r8 * 8 + k
                    rows[rq, a] = jnp.maximum(
                        rows[rq, a] + emb[r8, pl.ds(k * 16, 16)], 0.0)
                return vc

            lax.fori_loop(0, _K // 8, vbody, 0, unroll=2)

            @pl.when(i >= 1)
            def _():
                drain(ot[6], ot)

            @pl.when(i + 1 < _NCH)
            def _():
                fire(i + 1, ot)

            pltpu.async_copy(rows, accum.at[idst], semsc, add=True)

        fire(0, sets[0])

        def pair(k, carry):
            step(2 * k, sets[0], sets[1])
            step(2 * k + 1, sets[1], sets[0])
            return carry

        lax.fori_loop(0, _NCH // 2, pair, 0)
        drain(sets[1][6], sets[1])
        plsc.subcore_barrier()

        w0 = s * 3128
        out_g = outs[g]

        @pl.when(s < 15)
        def _():
            pltpu.sync_copy(
                accum.at[pl.ds(w0, 3128)],
                out_g.at[pl.ds(c * N_NODES + w0, 3128), pl.ds(0, _C)])

        @pl.when(s == 15)
        def _():
            pltpu.sync_copy(
                accum.at[pl.ds(w0, 3080)],
                out_g.at[pl.ds(c * N_NODES + w0, 3080), pl.ds(0, _C)])

        plsc.subcore_barrier()


def _sc_aggr(src2d, dst2d, xs, es, n_groups):
    mesh = plsc.VectorSubcoreMesh(core_axis_name="c", subcore_axis_name="s")
    f = pl.kernel(
        functools.partial(_sc_aggr_body, n_groups),
        out_type=[jax.ShapeDtypeStruct((2 * N_NODES, 128), jnp.float32)
                  for _ in range(n_groups)],
        mesh=mesh,
        scratch_types=[
            pltpu.VMEM((_K,), jnp.int32),
            pltpu.VMEM((_K,), jnp.int32),
            pltpu.VMEM((_K,), jnp.int32),
            pltpu.VMEM((_K,), jnp.int32),
            pltpu.VMEM((_K, _C), jnp.float32),
            pltpu.VMEM((_K, _C), jnp.float32),
            pltpu.VMEM((_K // 8, 128), jnp.float32),
            pltpu.VMEM((_K // 8, 128), jnp.float32),
            pltpu.VMEM((1024, _C), jnp.float32),
            pltpu.VMEM_SHARED((_ACC_ROWS, _C), jnp.float32),
            pltpu.SemaphoreType.DMA,
            pltpu.SemaphoreType.DMA,
            pltpu.SemaphoreType.DMA,
            pltpu.SemaphoreType.DMA,
            pltpu.SemaphoreType.DMA,
            pltpu.SemaphoreType.DMA,
        ],
        compiler_params=pltpu.CompilerParams(use_tc_tiling_on_sc=False),
    )
    return f(src2d, dst2d, *xs, *es)


def _mlp_body(n_groups, x_ref, *refs, act, n_groups_next):
    parts = refs[:2 * n_groups]
    (w1_ref, b1_ref, w2_ref, b2_ref, g_ref, bt_ref, eps_ref) = \
        refs[2 * n_groups:2 * n_groups + 7]
    o_ref = refs[2 * n_groups + 7]
    o2_ref = refs[2 * n_groups + 8] if n_groups_next else None
    aggr = jnp.concatenate(
        [parts[2 * g][:, :_C] + parts[2 * g + 1][:, :_C]
         for g in range(n_groups)],
        axis=1)
    h0 = (1.0 + eps_ref[0, 0]) * x_ref[...] + aggr
    z = jnp.dot(h0, w1_ref[...], preferred_element_type=jnp.float32) + b1_ref[...]
    z = jnp.maximum(z, 0.0)
    h = jnp.dot(z, w2_ref[...], preferred_element_type=jnp.float32) + b2_ref[...]
    mu = jnp.mean(h, axis=-1, keepdims=True)
    var = jnp.mean((h - mu) ** 2, axis=-1, keepdims=True)
    h = (h - mu) * lax.rsqrt(var + 1e-5) * g_ref[...] + bt_ref[...]
    if act:
        h = jnp.maximum(h, 0.0)
    o_ref[...] = h
    if o2_ref is not None:
        br = h.shape[0]
        o2_ref[...] = jnp.transpose(
            h.reshape(br, n_groups_next, _C), (1, 0, 2))


def _mlp(xpad, parts, w1t_pad, b1, w2t, b2, gamma, beta, eps, act,
         n_groups_next):
    n, dp = xpad.shape
    n_groups = len(parts)
    dout = w2t.shape[0]
    br = 2000
    nblk = n // br
    out_shape = [jax.ShapeDtypeStruct((n, dout), jnp.float32)]
    out_specs = [pl.BlockSpec((br, dout), lambda i: (i, 0))]
    if n_groups_next:
        out_shape.append(
            jax.ShapeDtypeStruct((n_groups_next, n, _C), jnp.float32))
        out_specs.append(
            pl.BlockSpec((n_groups_next, br, _C), lambda i: (0, i, 0)))
    part_specs = []
    part_args = []
    for p in parts:
        part_specs.append(pl.BlockSpec((br, 128), lambda i: (i, 0)))
        part_specs.append(
            pl.BlockSpec((br, 128), lambda i, _nb=nblk: (_nb + i, 0)))
        part_args.extend([p, p])
    res = pl.pallas_call(
        functools.partial(_mlp_body, n_groups, act=act,
                          n_groups_next=n_groups_next),
        grid=(nblk,),
        in_specs=[
            pl.BlockSpec((br, dp), lambda i: (i, 0)),
            *part_specs,
            pl.BlockSpec((dp, dout), lambda i: (0, 0)),
            pl.BlockSpec((1, dout), lambda i: (0, 0)),
            pl.BlockSpec((dout, dout), lambda i: (0, 0)),
            pl.BlockSpec((1, dout), lambda i: (0, 0)),
            pl.BlockSpec((1, dout), lambda i: (0, 0)),
            pl.BlockSpec((1, dout), lambda i: (0, 0)),
            pl.BlockSpec((1, 1), lambda i: (0, 0), memory_space=pltpu.SMEM),
        ],
        out_specs=out_specs,
        out_shape=out_shape,
    )(xpad, *part_args, w1t_pad, b1[None, :], w2t, b2[None, :],
      gamma[None, :], beta[None, :], eps.reshape(1, 1))
    return res if n_groups_next else (res[0], None)


def kernel(x, edge_index, edge_attr, batch,
           l1_Wel, l1_bel, l1_W1, l1_b1, l1_W2, l1_b2, l1_gamma, l1_beta, l1_eps,
           l2_Wel, l2_bel, l2_W1, l2_b1, l2_W2, l2_b2, l2_gamma, l2_beta, l2_eps,
           l3_Wel, l3_bel, l3_W1, l3_b1, l3_W2, l3_b2, l3_gamma, l3_beta, l3_eps):
    layers = [
        (l1_Wel, l1_bel, l1_W1, l1_b1, l1_W2, l1_b2, l1_gamma, l1_beta, l1_eps, True),
        (l2_Wel, l2_bel, l2_W1, l2_b1, l2_W2, l2_b2, l2_gamma, l2_beta, l2_eps, True),
        (l3_Wel, l3_bel, l3_W1, l3_b1, l3_W2, l3_b2, l3_gamma, l3_beta, l3_eps, False),
    ]
    pad_e = EDGES_PAD - N_EDGES
    src2d = jnp.pad(edge_index[0], (0, pad_e))
    dst2d = jnp.pad(edge_index[1], (0, pad_e), constant_values=N_NODES)
    ea8w = jnp.pad(edge_attr, ((0, pad_e), (0, 2))).reshape(-1, 64)

    din1, dp1, p1, _ = _LAYER_DIMS[0]
    xpad = jnp.pad(x, ((0, 0), (0, dp1 - din1)))
    xsplit = jnp.transpose(xpad.reshape(N_NODES, p1, _C), (1, 0, 2))

    h = xpad
    for li, ((din, dp, ng, dout),
             (wel, bel, w1, b1, w2, b2, gamma, beta, eps, act)) in enumerate(
            zip(_LAYER_DIMS, layers)):
        welt_pad = jnp.pad(wel.T, ((0, 2), (0, dp - din)))
        bel_pad = jnp.pad(bel, (0, dp - din))[None, :]
        w1t_pad = jnp.pad(w1.T, ((0, dp - din), (0, 0)))
        eemb = _eemb(ea8w, welt_pad, bel_pad, ng)
        parts = _sc_aggr(src2d, dst2d,
                         [xsplit[g] for g in range(ng)],
                         [eemb[g] for g in range(ng)], ng)
        ng_next = _LAYER_DIMS[li + 1][2] if li < 2 else 0
        h, xsplit = _mlp(h, parts, w1t_pad, b1, w2.T, b2, gamma, beta, eps,
                         act, ng_next)
    return (h, batch)
